# baseline (speedup 1.0000x reference)
import functools

import jax
import jax.numpy as jnp
from jax import lax
from jax.experimental import pallas as pl
from jax.experimental.pallas import tpu as pltpu
from jax.experimental.pallas import tpu_sc as plsc

N = 10000
NP = 10240
E = 320000
D = 128
K = 512
NC = 2
NS = 16
L = 16
CH = 80

W1 = NC * NS
EP1 = E // W1
NCH1 = EP1 // CH
EP3 = E // NS
NCH3 = EP3 // CH

@functools.cache
def _mesh():
    return plsc.VectorSubcoreMesh(
        core_axis_name="c", subcore_axis_name="s",
        num_cores=NC, num_subcores=NS)


def _zero_rows(ref, nrows, ncols=D):
    zeros = jnp.zeros((L,), jnp.float32)

    def body(i, _):
        for j in range(ncols // L):
            ref[i, pl.ds(j * L, L)] = zeros
        return 0

    lax.fori_loop(0, nrows, body, 0)


def _deg_body(edges_hbm, out_hbm, idx_v, hist_v, rowidx_v, wb_v, shared_h):
    cid = lax.axis_index("c")
    sid = lax.axis_index("s")

    _zero_rows(hist_v, NP // D)

    @pl.when(sid == 0)
    def _():
        pltpu.sync_copy(hist_v, shared_h)

    for j in range(CH // L):
        rowidx_v[pl.ds(j * L, L)] = (
            lax.broadcasted_iota(jnp.int32, (L,), 0) + (j * L))

    plsc.subcore_barrier()

    pltpu.sync_copy(edges_hbm.at[cid, sid], idx_v)
    ones = jnp.ones((L,), jnp.float32)

    def body(i, _):
        for j in range(CH // L):
            idx = idx_v[i, pl.ds(j * L, L)]
            row = lax.shift_right_logical(idx, 7)
            col = lax.bitwise_and(idx, 127)
            plsc.addupdate_scatter(hist_v, [row, col], ones)
        return 0

    lax.fori_loop(0, NCH3, body, 0)

    pltpu.sync_copy(hist_v, shared_h.at[rowidx_v], add=True)
    plsc.subcore_barrier()

    @pl.when(sid < (NP // D) // 8)
    def _():
        pltpu.sync_copy(shared_h.at[pl.ds(sid * 8, 8)], wb_v)
        pltpu.sync_copy(wb_v, out_hbm.at[cid, pl.ds(sid * 8, 8)])


@functools.cache
def _deg_kernel():
    return pl.kernel(
        _deg_body,
        out_type=jax.ShapeDtypeStruct((NC, NP // D, D), jnp.float32),
        mesh=_mesh(),
        scratch_types=[
            pltpu.VMEM((NCH3, CH), jnp.int32),
            pltpu.VMEM((NP // D, D), jnp.float32),
            pltpu.VMEM((CH,), jnp.int32),
            pltpu.VMEM((8, D), jnp.float32),
            pltpu.VMEM_SHARED((NP // D, D), jnp.float32),
        ],
        compiler_params=pltpu.CompilerParams(needs_layout_passes=False),
    )


BB = 10
NBLK = NCH3 // BB


def _edge_body(h_hbm, edges_hbm, out_hbm, eidx_v, rows_a, rows_b, zero_v,
               shared_agg, sem_a, sem_b):
    cid = lax.axis_index("c")
    sid = lax.axis_index("s")

    _zero_rows(zero_v, 8)
    rows_per = NP // NS
    for k in range(rows_per // 8):
        pltpu.sync_copy(zero_v, shared_agg.at[pl.ds(sid * rows_per + k * 8, 8)])

    plsc.subcore_barrier()

    off = jnp.full((L,), cid * NP, jnp.int32)

    def load_block(blk):
        buf = blk % 2
        pltpu.sync_copy(edges_hbm.at[0, sid, blk], eidx_v.at[buf, 0])
        pltpu.sync_copy(edges_hbm.at[1, sid, blk], eidx_v.at[buf, 1])

        def adjust(i, _):
            for j in range(CH // L):
                sl = pl.ds(j * L, L)
                eidx_v[buf, 0, i, sl] = eidx_v[buf, 0, i, sl] + off
            return 0

        lax.fori_loop(0, BB, adjust, 0)

    def gather(c, rows, sem):
        return pltpu.async_copy(
            h_hbm.at[eidx_v.at[(c // BB) % 2, 0, c % BB]], rows, sem)

    def scatter(c, rows):
        pltpu.sync_copy(
            rows, shared_agg.at[eidx_v.at[(c // BB) % 2, 1, c % BB]], add=True)

    load_block(0)
    gather(0, rows_a, sem_a)

    def pair(k, _):
        c0 = 2 * k
        gather(c0 + 1, rows_b, sem_b)
        pltpu.make_async_copy(h_hbm.at[eidx_v.at[0, 0, 0]], rows_a, sem_a).wait()
        scatter(c0, rows_a)

        @pl.when(k < NCH3 // 2 - 1)
        def _():
            @pl.when((c0 + 2) % BB == 0)
            def _():
                load_block((c0 + 2) // BB)
            gather(c0 + 2, rows_a, sem_a)

        pltpu.make_async_copy(h_hbm.at[eidx_v.at[0, 0, 0]], rows_b, sem_b).wait()
        scatter(c0 + 1, rows_b)
        return 0

    lax.fori_loop(0, NCH3 // 2, pair, 0)

    plsc.subcore_barrier()

    sl = pl.ds(sid * rows_per, rows_per)
    pltpu.sync_copy(shared_agg.at[sl], out_hbm.at[cid, sl])


@functools.cache
def _edge_kernel():
    return pl.kernel(
        _edge_body,
        out_type=jax.ShapeDtypeStruct((NC, NP, D), jnp.float32),
        mesh=_mesh(),
        scratch_types=[
            pltpu.VMEM((2, 2, BB, CH), jnp.int32),
            pltpu.VMEM((CH, D), jnp.float32),
            pltpu.VMEM((CH, D), jnp.float32),
            pltpu.VMEM((8, D), jnp.float32),
            pltpu.VMEM_SHARED((NP, D), jnp.float32),
            pltpu.SemaphoreType.DMA,
            pltpu.SemaphoreType.DMA,
        ],
    )


RB2 = 2048


def _mm_body(x_ref, cx_ref, w_ref, b_ref, degs_ref, degr_ref,
             h_ref, scale_r_ref):
    i = pl.program_id(0)
    scale_s = lax.rsqrt(jnp.maximum(degs_ref[...], 1.0))
    scale_r_ref[...] = lax.rsqrt(jnp.maximum(degr_ref[...], 1.0))

    @pl.when(i == 0)
    def _():
        h = jnp.dot(x_ref[...], w_ref[...], preferred_element_type=jnp.float32)
        h_ref[...] = (h + b_ref[...]) * scale_s

    @pl.when(i == 1)
    def _():
        h = jnp.dot(cx_ref[...], w_ref[...],
                    preferred_element_type=jnp.float32)
        h_ref[...] = (h + b_ref[...]) * scale_s


def _mm_kernel(x, c_x, W, b2, deg_s, deg_r):
    nj = NP // RB2
    return pl.pallas_call(
        _mm_body,
        grid=(2, nj),
        in_specs=[
            pl.BlockSpec((RB2, D), lambda i, j: (j, 0)),
            pl.BlockSpec((RB2, D), lambda i, j: (j, 0)),
            pl.BlockSpec((D, D), lambda i, j: (0, 0)),
            pl.BlockSpec((1, D), lambda i, j: (0, 0)),
            pl.BlockSpec((RB2, 1), lambda i, j: (j, 0)),
            pl.BlockSpec((RB2, 1), lambda i, j: (j, 0)),
        ],
        out_specs=[
            pl.BlockSpec((RB2, D), lambda i, j, nj=nj: (i * nj + j, 0)),
            pl.BlockSpec((RB2, 1), lambda i, j: (j, 0)),
        ],
        out_shape=[
            jax.ShapeDtypeStruct((2 * NP, D), jnp.float32),
            jax.ShapeDtypeStruct((NP, 1), jnp.float32),
        ],
    )(x, c_x, W, b2, deg_s, deg_r)


SELU_L = 1.0507009873554804934193349852946
SELU_A = 1.6732632423543772848170429916717


def _selu(a):
    return SELU_L * jnp.where(a > 0, a, SELU_A * (jnp.exp(a) - 1.0))


RB3 = 1024
NB3 = NP // RB3


def _head_body(a1_ref, ap_ref, scale_r_ref, w_blT_ref, centersT_ref,
               emb_ref, l1_ref, l2_ref, rep_ref, loss_ref,
               rowsum_s, v_s, csq_s, runmin_s, runarg_s, loss_s):
    p = pl.program_id(0)
    j = pl.program_id(1)

    scale_r = scale_r_ref[...]
    n1 = _selu(a1_ref[0] * scale_r)
    nrm = jnp.sqrt(jnp.sum(n1 * n1, axis=1, keepdims=True))
    emb = n1 / (nrm + 1e-12)
    emb_ref[...] = emb

    @pl.when(jnp.logical_and(p == 0, j == 0))
    def _():
        csq_s[...] = jnp.sum(centersT_ref[...] * centersT_ref[...], axis=0,
                             keepdims=True)
        rowsum_s[...] = jnp.zeros_like(rowsum_s)
        runmin_s[...] = jnp.full_like(runmin_s, jnp.inf)
        runarg_s[...] = jnp.zeros_like(runarg_s)
        loss_s[...] = jnp.zeros_like(loss_s)

    @pl.when(p == 0)
    def _():
        rowsum_s[...] += jnp.sum(n1, axis=0, keepdims=True)
        l1_ref[...] = jnp.zeros_like(l1_ref)
        l2_ref[...] = jnp.zeros_like(l2_ref)

        esq = jnp.sum(emb * emb, axis=1, keepdims=True)
        prod = jnp.dot(emb, centersT_ref[...],
                       preferred_element_type=jnp.float32)
        sq = esq + csq_s[...] - 2.0 * prod
        dists = jnp.sqrt(jnp.maximum(sq, 1e-12))

        row_local = lax.broadcasted_iota(jnp.int32, (RB3, K), 0)
        row_global = row_local + j * RB3
        valid = row_global < N
        dists_m = jnp.where(valid, dists, jnp.inf)

        dmin = jnp.min(dists_m, axis=1, keepdims=True)
        dmin = jnp.where(valid[:, :1], dmin, 0.0)
        loss_s[...] += jnp.sum(dmin).reshape(1, 1)

        bmin = jnp.min(dists_m, axis=0, keepdims=True)
        idxm = jnp.where(dists_m == bmin, row_global, 2**30)
        barg = jnp.min(idxm, axis=0, keepdims=True)
        better = bmin < runmin_s[...]
        runarg_s[...] = jnp.where(better, barg, runarg_s[...])
        runmin_s[...] = jnp.minimum(bmin, runmin_s[...])

        @pl.when(j == NB3 - 1)
        def _():
            rep_ref[...] = runarg_s[...]
            loss_ref[...] = loss_s[...]

    @pl.when(p == 1)
    def _():
        @pl.when(j == 0)
        def _():
            summary = 1.0 / (1.0 + jnp.exp(-rowsum_s[...] / N))
            v_s[...] = jnp.dot(summary, w_blT_ref[...],
                               preferred_element_type=jnp.float32)

        n2 = _selu(ap_ref[0] * scale_r)
        v = v_s[...]
        l1_ref[...] = jnp.sum(n1 * v, axis=1, keepdims=True)
        l2_ref[...] = jnp.sum(n2 * v, axis=1, keepdims=True)


def _head_kernel(agg, scale_r, w_blT, centersT):
    return pl.pallas_call(
        _head_body,
        grid=(2, NB3),
        in_specs=[
            pl.BlockSpec((1, RB3, D), lambda p, j: (0, j, 0)),
            pl.BlockSpec((1, RB3, D), lambda p, j: (p, j, 0)),
            pl.BlockSpec((RB3, 1), lambda p, j: (j, 0)),
            pl.BlockSpec((D, D), lambda p, j: (0, 0)),
            pl.BlockSpec((D, K), lambda p, j: (0, 0)),
        ],
        out_specs=[
            pl.BlockSpec((RB3, D), lambda p, j: (j, 0)),
            pl.BlockSpec((RB3, 1), lambda p, j: (j, 0)),
            pl.BlockSpec((RB3, 1), lambda p, j: (j, 0)),
            pl.BlockSpec((1, K), lambda p, j: (0, 0)),
            pl.BlockSpec((1, 1), lambda p, j: (0, 0)),
        ],
        out_shape=[
            jax.ShapeDtypeStruct((N, D), jnp.float32),
            jax.ShapeDtypeStruct((N, 1), jnp.float32),
            jax.ShapeDtypeStruct((N, 1), jnp.float32),
            jax.ShapeDtypeStruct((1, K), jnp.int32),
            jax.ShapeDtypeStruct((1, 1), jnp.float32),
        ],
        scratch_shapes=[
            pltpu.VMEM((1, D), jnp.float32),
            pltpu.VMEM((1, D), jnp.float32),
            pltpu.VMEM((1, K), jnp.float32),
            pltpu.VMEM((1, K), jnp.float32),
            pltpu.VMEM((1, K), jnp.int32),
            pltpu.VMEM((1, 1), jnp.float32),
        ],
    )(agg, agg, scale_r, w_blT, centersT)


def kernel(x, c_x, edge_index, W, b, w_bl, centers):
    ei = edge_index.astype(jnp.int32)
    e3 = ei.reshape(2, NS, NBLK, BB, CH)

    e_deg = ei.reshape(2, NS, NCH3, CH)
    degp = _deg_kernel()(e_deg)

    h, scale_r = _mm_kernel(x, c_x, W, b.reshape(1, D),
                            degp[0].reshape(NP, 1), degp[1].reshape(NP, 1))
    agg = _edge_kernel()(h, e3)

    emb_p, l1, l2, rep, loss = _head_kernel(
        agg, scale_r, w_bl.T, centers.T)

    emb = emb_p
    logits = jnp.concatenate([l1[:, 0], l2[:, 0]])
    rep_ids = rep[0]
    cluster_loss = loss[0, 0]
    return (emb, centers, rep_ids, cluster_loss, logits)


# speedup vs baseline: 1.0354x; 1.0354x over previous
import functools

import jax
import jax.numpy as jnp
from jax import lax
from jax.experimental import pallas as pl
from jax.experimental.pallas import tpu as pltpu
from jax.experimental.pallas import tpu_sc as plsc

N = 10000
NP = 10240
E = 320000
D = 128
K = 512
NC = 2
NS = 16
L = 16
CH = 80

W1 = NC * NS
EP1 = E // W1
NCH1 = EP1 // CH
EP3 = E // NS
NCH3 = EP3 // CH

@functools.cache
def _mesh():
    return plsc.VectorSubcoreMesh(
        core_axis_name="c", subcore_axis_name="s",
        num_cores=NC, num_subcores=NS)


def _zero_rows(ref, nrows, ncols=D):
    zeros = jnp.zeros((L,), jnp.float32)

    def body(i, _):
        for j in range(ncols // L):
            ref[i, pl.ds(j * L, L)] = zeros
        return 0

    lax.fori_loop(0, nrows, body, 0)


def _deg_body(edges_hbm, out_hbm, idx_v, hist_v, rowidx_v, wb_v, shared_h):
    cid = lax.axis_index("c")
    sid = lax.axis_index("s")

    _zero_rows(hist_v, NP // D)

    @pl.when(sid == 0)
    def _():
        pltpu.sync_copy(hist_v, shared_h)

    for j in range(CH // L):
        rowidx_v[pl.ds(j * L, L)] = (
            lax.broadcasted_iota(jnp.int32, (L,), 0) + (j * L))

    plsc.subcore_barrier()

    pltpu.sync_copy(edges_hbm.at[cid, sid], idx_v)
    ones = jnp.ones((L,), jnp.float32)

    def body(i, _):
        for j in range(CH // L):
            idx = idx_v[i, pl.ds(j * L, L)]
            row = lax.shift_right_logical(idx, 7)
            col = lax.bitwise_and(idx, 127)
            plsc.addupdate_scatter(hist_v, [row, col], ones)
        return 0

    lax.fori_loop(0, NCH3, body, 0)

    pltpu.sync_copy(hist_v, shared_h.at[rowidx_v], add=True)
    plsc.subcore_barrier()

    @pl.when(sid < (NP // D) // 8)
    def _():
        pltpu.sync_copy(shared_h.at[pl.ds(sid * 8, 8)], wb_v)
        pltpu.sync_copy(wb_v, out_hbm.at[cid, pl.ds(sid * 8, 8)])


@functools.cache
def _deg_kernel():
    return pl.kernel(
        _deg_body,
        out_type=jax.ShapeDtypeStruct((NC, NP // D, D), jnp.float32),
        mesh=_mesh(),
        scratch_types=[
            pltpu.VMEM((NCH3, CH), jnp.int32),
            pltpu.VMEM((NP // D, D), jnp.float32),
            pltpu.VMEM((CH,), jnp.int32),
            pltpu.VMEM((8, D), jnp.float32),
            pltpu.VMEM_SHARED((NP // D, D), jnp.float32),
        ],
        compiler_params=pltpu.CompilerParams(needs_layout_passes=False),
    )


ECH = 40
ENCH = EP3 // ECH
NRING = 4
LA = NRING - 1
BB = 10
NBLK = ENCH // BB


def _edge_body(h_hbm, edges_hbm, out_hbm, eidx_v, r0, r1, r2, r3,
               shared_agg, s0, s1, s2, s3):
    cid = lax.axis_index("c")
    sid = lax.axis_index("s")
    rows = [r0, r1, r2, r3]
    sems = [s0, s1, s2, s3]

    _zero_rows(r0, 8)
    rows_per = NP // NS
    for k in range(rows_per // 8):
        pltpu.sync_copy(
            r0.at[pl.ds(0, 8)],
            shared_agg.at[pl.ds(sid * rows_per + k * 8, 8)])

    plsc.subcore_barrier()

    off = jnp.full((L,), cid * NP, jnp.int32)

    def load_block(blk):
        buf = blk % 2
        pltpu.sync_copy(edges_hbm.at[0, sid, blk], eidx_v.at[buf, 0])
        pltpu.sync_copy(edges_hbm.at[1, sid, blk], eidx_v.at[buf, 1])

        def adjust(i, _):
            for j in range(ECH // L):
                sl = pl.ds(j * L, L)
                eidx_v[buf, 0, i, sl] = eidx_v[buf, 0, i, sl] + off
            return 0

        lax.fori_loop(0, BB, adjust, 0)

    def gather(c, s):
        pltpu.async_copy(
            h_hbm.at[eidx_v.at[(c // BB) % 2, 0, c % BB]], rows[s], sems[s])

    def scatter(c, s):
        pltpu.sync_copy(
            rows[s], shared_agg.at[eidx_v.at[(c // BB) % 2, 1, c % BB]],
            add=True)

    load_block(0)
    for c in range(LA):
        gather(c, c % NRING)

    def quad(k, _):
        for s in range(NRING):
            c = NRING * k + s

            @pl.when(c + LA < ENCH)
            def _():
                @pl.when((c + LA) % BB == 0)
                def _():
                    load_block((c + LA) // BB)
                gather(c + LA, (s + LA) % NRING)

            pltpu.make_async_copy(
                h_hbm.at[eidx_v.at[0, 0, 0]], rows[s], sems[s]).wait()
            scatter(c, s)
        return 0

    lax.fori_loop(0, ENCH // NRING, quad, 0)

    plsc.subcore_barrier()

    sl = pl.ds(sid * rows_per, rows_per)
    pltpu.sync_copy(shared_agg.at[sl], out_hbm.at[cid, sl])


@functools.cache
def _edge_kernel():
    return pl.kernel(
        _edge_body,
        out_type=jax.ShapeDtypeStruct((NC, NP, D), jnp.float32),
        mesh=_mesh(),
        scratch_types=[
            pltpu.VMEM((2, 2, BB, ECH), jnp.int32),
            pltpu.VMEM((ECH, D), jnp.float32),
            pltpu.VMEM((ECH, D), jnp.float32),
            pltpu.VMEM((ECH, D), jnp.float32),
            pltpu.VMEM((ECH, D), jnp.float32),
            pltpu.VMEM_SHARED((NP, D), jnp.float32),
            pltpu.SemaphoreType.DMA,
            pltpu.SemaphoreType.DMA,
            pltpu.SemaphoreType.DMA,
            pltpu.SemaphoreType.DMA,
        ],
    )


RB2 = 2048


def _mm_body(x_ref, cx_ref, w_ref, b_ref, degs_ref, degr_ref,
             h_ref, scale_r_ref):
    i = pl.program_id(0)
    scale_s = lax.rsqrt(jnp.maximum(degs_ref[...], 1.0))
    scale_r_ref[...] = lax.rsqrt(jnp.maximum(degr_ref[...], 1.0))

    @pl.when(i == 0)
    def _():
        h = jnp.dot(x_ref[...], w_ref[...], preferred_element_type=jnp.float32)
        h_ref[...] = (h + b_ref[...]) * scale_s

    @pl.when(i == 1)
    def _():
        h = jnp.dot(cx_ref[...], w_ref[...],
                    preferred_element_type=jnp.float32)
        h_ref[...] = (h + b_ref[...]) * scale_s


def _mm_kernel(x, c_x, W, b2, deg_s, deg_r):
    nj = NP // RB2
    return pl.pallas_call(
        _mm_body,
        grid=(2, nj),
        in_specs=[
            pl.BlockSpec((RB2, D), lambda i, j: (j, 0)),
            pl.BlockSpec((RB2, D), lambda i, j: (j, 0)),
            pl.BlockSpec((D, D), lambda i, j: (0, 0)),
            pl.BlockSpec((1, D), lambda i, j: (0, 0)),
            pl.BlockSpec((RB2, 1), lambda i, j: (j, 0)),
            pl.BlockSpec((RB2, 1), lambda i, j: (j, 0)),
        ],
        out_specs=[
            pl.BlockSpec((RB2, D), lambda i, j, nj=nj: (i * nj + j, 0)),
            pl.BlockSpec((RB2, 1), lambda i, j: (j, 0)),
        ],
        out_shape=[
            jax.ShapeDtypeStruct((2 * NP, D), jnp.float32),
            jax.ShapeDtypeStruct((NP, 1), jnp.float32),
        ],
    )(x, c_x, W, b2, deg_s, deg_r)


SELU_L = 1.0507009873554804934193349852946
SELU_A = 1.6732632423543772848170429916717


def _selu(a):
    return SELU_L * jnp.where(a > 0, a, SELU_A * (jnp.exp(a) - 1.0))


RB3 = 1024
NB3 = NP // RB3


def _head_body(a1_ref, ap_ref, scale_r_ref, w_blT_ref, centersT_ref,
               emb_ref, l1_ref, l2_ref, rep_ref, loss_ref,
               rowsum_s, v_s, csq_s, runmin_s, runarg_s, loss_s):
    p = pl.program_id(0)
    j = pl.program_id(1)

    scale_r = scale_r_ref[...]
    n1 = _selu(a1_ref[0] * scale_r)
    nrm = jnp.sqrt(jnp.sum(n1 * n1, axis=1, keepdims=True))
    emb = n1 / (nrm + 1e-12)
    emb_ref[...] = emb

    @pl.when(jnp.logical_and(p == 0, j == 0))
    def _():
        csq_s[...] = jnp.sum(centersT_ref[...] * centersT_ref[...], axis=0,
                             keepdims=True)
        rowsum_s[...] = jnp.zeros_like(rowsum_s)
        runmin_s[...] = jnp.full_like(runmin_s, jnp.inf)
        runarg_s[...] = jnp.zeros_like(runarg_s)
        loss_s[...] = jnp.zeros_like(loss_s)

    @pl.when(p == 0)
    def _():
        rowsum_s[...] += jnp.sum(n1, axis=0, keepdims=True)
        l1_ref[...] = jnp.zeros_like(l1_ref)
        l2_ref[...] = jnp.zeros_like(l2_ref)

        esq = jnp.sum(emb * emb, axis=1, keepdims=True)
        prod = jnp.dot(emb, centersT_ref[...],
                       preferred_element_type=jnp.float32)
        sq = esq + csq_s[...] - 2.0 * prod
        dists = jnp.sqrt(jnp.maximum(sq, 1e-12))

        row_local = lax.broadcasted_iota(jnp.int32, (RB3, K), 0)
        row_global = row_local + j * RB3
        valid = row_global < N
        dists_m = jnp.where(valid, dists, jnp.inf)

        dmin = jnp.min(dists_m, axis=1, keepdims=True)
        dmin = jnp.where(valid[:, :1], dmin, 0.0)
        loss_s[...] += jnp.sum(dmin).reshape(1, 1)

        bmin = jnp.min(dists_m, axis=0, keepdims=True)
        idxm = jnp.where(dists_m == bmin, row_global, 2**30)
        barg = jnp.min(idxm, axis=0, keepdims=True)
        better = bmin < runmin_s[...]
        runarg_s[...] = jnp.where(better, barg, runarg_s[...])
        runmin_s[...] = jnp.minimum(bmin, runmin_s[...])

        @pl.when(j == NB3 - 1)
        def _():
            rep_ref[...] = runarg_s[...]
            loss_ref[...] = loss_s[...]

    @pl.when(p == 1)
    def _():
        @pl.when(j == 0)
        def _():
            summary = 1.0 / (1.0 + jnp.exp(-rowsum_s[...] / N))
            v_s[...] = jnp.dot(summary, w_blT_ref[...],
                               preferred_element_type=jnp.float32)

        n2 = _selu(ap_ref[0] * scale_r)
        v = v_s[...]
        l1_ref[...] = jnp.sum(n1 * v, axis=1, keepdims=True)
        l2_ref[...] = jnp.sum(n2 * v, axis=1, keepdims=True)


def _head_kernel(agg, scale_r, w_blT, centersT):
    return pl.pallas_call(
        _head_body,
        grid=(2, NB3),
        in_specs=[
            pl.BlockSpec((1, RB3, D), lambda p, j: (0, j, 0)),
            pl.BlockSpec((1, RB3, D), lambda p, j: (p, j, 0)),
            pl.BlockSpec((RB3, 1), lambda p, j: (j, 0)),
            pl.BlockSpec((D, D), lambda p, j: (0, 0)),
            pl.BlockSpec((D, K), lambda p, j: (0, 0)),
        ],
        out_specs=[
            pl.BlockSpec((RB3, D), lambda p, j: (j, 0)),
            pl.BlockSpec((RB3, 1), lambda p, j: (j, 0)),
            pl.BlockSpec((RB3, 1), lambda p, j: (j, 0)),
            pl.BlockSpec((1, K), lambda p, j: (0, 0)),
            pl.BlockSpec((1, 1), lambda p, j: (0, 0)),
        ],
        out_shape=[
            jax.ShapeDtypeStruct((N, D), jnp.float32),
            jax.ShapeDtypeStruct((N, 1), jnp.float32),
            jax.ShapeDtypeStruct((N, 1), jnp.float32),
            jax.ShapeDtypeStruct((1, K), jnp.int32),
            jax.ShapeDtypeStruct((1, 1), jnp.float32),
        ],
        scratch_shapes=[
            pltpu.VMEM((1, D), jnp.float32),
            pltpu.VMEM((1, D), jnp.float32),
            pltpu.VMEM((1, K), jnp.float32),
            pltpu.VMEM((1, K), jnp.float32),
            pltpu.VMEM((1, K), jnp.int32),
            pltpu.VMEM((1, 1), jnp.float32),
        ],
    )(agg, agg, scale_r, w_blT, centersT)


def kernel(x, c_x, edge_index, W, b, w_bl, centers):
    ei = edge_index.astype(jnp.int32)
    e3 = ei.reshape(2, NS, NBLK, BB, ECH)

    e_deg = ei.reshape(2, NS, NCH3, CH)
    degp = _deg_kernel()(e_deg)

    h, scale_r = _mm_kernel(x, c_x, W, b.reshape(1, D),
                            degp[0].reshape(NP, 1), degp[1].reshape(NP, 1))
    agg = _edge_kernel()(h, e3)

    emb_p, l1, l2, rep, loss = _head_kernel(
        agg, scale_r, w_bl.T, centers.T)

    emb = emb_p
    logits = jnp.concatenate([l1[:, 0], l2[:, 0]])
    rep_ids = rep[0]
    cluster_loss = loss[0, 0]
    return (emb, centers, rep_ids, cluster_loss, logits)


# speedup vs baseline: 1.1016x; 1.0639x over previous
import functools

import jax
import jax.numpy as jnp
from jax import lax
from jax.experimental import pallas as pl
from jax.experimental.pallas import tpu as pltpu
from jax.experimental.pallas import tpu_sc as plsc

N = 10000
NP = 10240
E = 320000
D = 128
K = 512
NC = 2
NS = 16
L = 16
CH = 80

W1 = NC * NS
EP1 = E // W1
NCH1 = EP1 // CH
EP3 = E // NS
NCH3 = EP3 // CH

@functools.cache
def _mesh():
    return plsc.VectorSubcoreMesh(
        core_axis_name="c", subcore_axis_name="s",
        num_cores=NC, num_subcores=NS)


def _zero_rows(ref, nrows, ncols=D):
    zeros = jnp.zeros((L,), jnp.float32)

    def body(i, _):
        for j in range(ncols // L):
            ref[i, pl.ds(j * L, L)] = zeros
        return 0

    lax.fori_loop(0, nrows, body, 0)


def _deg_body(edges_hbm, out_hbm, idx_v, hist_v, rowidx_v, wb_v, shared_h):
    cid = lax.axis_index("c")
    sid = lax.axis_index("s")

    _zero_rows(hist_v, NP // D)

    @pl.when(sid == 0)
    def _():
        pltpu.sync_copy(hist_v, shared_h)

    for j in range(CH // L):
        rowidx_v[pl.ds(j * L, L)] = (
            lax.broadcasted_iota(jnp.int32, (L,), 0) + (j * L))

    plsc.subcore_barrier()

    pltpu.sync_copy(edges_hbm.at[cid, sid], idx_v)
    ones = jnp.ones((L,), jnp.float32)

    def body(i, _):
        for j in range(CH // L):
            idx = idx_v[i, pl.ds(j * L, L)]
            row = lax.shift_right_logical(idx, 7)
            col = lax.bitwise_and(idx, 127)
            plsc.addupdate_scatter(hist_v, [row, col], ones)
        return 0

    lax.fori_loop(0, NCH3, body, 0)

    pltpu.sync_copy(hist_v, shared_h.at[rowidx_v], add=True)
    plsc.subcore_barrier()

    @pl.when(sid < (NP // D) // 8)
    def _():
        pltpu.sync_copy(shared_h.at[pl.ds(sid * 8, 8)], wb_v)
        pltpu.sync_copy(wb_v, out_hbm.at[cid, pl.ds(sid * 8, 8)])


@functools.cache
def _deg_kernel():
    return pl.kernel(
        _deg_body,
        out_type=jax.ShapeDtypeStruct((NC, NP // D, D), jnp.float32),
        mesh=_mesh(),
        scratch_types=[
            pltpu.VMEM((NCH3, CH), jnp.int32),
            pltpu.VMEM((NP // D, D), jnp.float32),
            pltpu.VMEM((CH,), jnp.int32),
            pltpu.VMEM((8, D), jnp.float32),
            pltpu.VMEM_SHARED((NP // D, D), jnp.float32),
        ],
        compiler_params=pltpu.CompilerParams(needs_layout_passes=False),
    )


ECH = 32
ENCH = EP3 // ECH
NRING = 5
LA = NRING - 1
BB = 25
NBLK = ENCH // BB


def _edge_body(h_hbm, edges_hbm, out_hbm, eidx_v, r0, r1, r2, r3, r4,
               shared_agg, s0, s1, s2, s3, s4):
    cid = lax.axis_index("c")
    sid = lax.axis_index("s")
    rows = [r0, r1, r2, r3, r4]
    sems = [s0, s1, s2, s3, s4]

    _zero_rows(r0, 8)
    rows_per = NP // NS
    for k in range(rows_per // 8):
        pltpu.sync_copy(
            r0.at[pl.ds(0, 8)],
            shared_agg.at[pl.ds(sid * rows_per + k * 8, 8)])

    plsc.subcore_barrier()

    off = jnp.full((L,), cid * NP, jnp.int32)

    def load_block(blk):
        buf = blk % 2
        pltpu.sync_copy(edges_hbm.at[0, sid, blk], eidx_v.at[buf, 0])
        pltpu.sync_copy(edges_hbm.at[1, sid, blk], eidx_v.at[buf, 1])

        def adjust(i, _):
            for j in range(ECH // L):
                sl = pl.ds(j * L, L)
                eidx_v[buf, 0, i, sl] = eidx_v[buf, 0, i, sl] + off
            return 0

        lax.fori_loop(0, BB, adjust, 0)

    def gather(c, s):
        pltpu.async_copy(
            h_hbm.at[eidx_v.at[(c // BB) % 2, 0, c % BB]], rows[s], sems[s])

    def scatter(c, s):
        pltpu.sync_copy(
            rows[s], shared_agg.at[eidx_v.at[(c // BB) % 2, 1, c % BB]],
            add=True)

    load_block(0)
    for c in range(LA):
        gather(c, c % NRING)

    def quad(k, _):
        for s in range(NRING):
            c = NRING * k + s

            @pl.when(c + LA < ENCH)
            def _():
                @pl.when((c + LA) % BB == 0)
                def _():
                    load_block((c + LA) // BB)
                gather(c + LA, (s + LA) % NRING)

            pltpu.make_async_copy(
                h_hbm.at[eidx_v.at[0, 0, 0]], rows[s], sems[s]).wait()
            scatter(c, s)
        return 0

    lax.fori_loop(0, ENCH // NRING, quad, 0)

    plsc.subcore_barrier()

    sl = pl.ds(sid * rows_per, rows_per)
    pltpu.sync_copy(shared_agg.at[sl], out_hbm.at[cid, sl])


@functools.cache
def _edge_kernel():
    return pl.kernel(
        _edge_body,
        out_type=jax.ShapeDtypeStruct((NC, NP, D), jnp.float32),
        mesh=_mesh(),
        scratch_types=[
            pltpu.VMEM((2, 2, BB, ECH), jnp.int32),
            pltpu.VMEM((ECH, D), jnp.float32),
            pltpu.VMEM((ECH, D), jnp.float32),
            pltpu.VMEM((ECH, D), jnp.float32),
            pltpu.VMEM((ECH, D), jnp.float32),
            pltpu.VMEM((ECH, D), jnp.float32),
            pltpu.VMEM_SHARED((NP, D), jnp.float32),
            pltpu.SemaphoreType.DMA,
            pltpu.SemaphoreType.DMA,
            pltpu.SemaphoreType.DMA,
            pltpu.SemaphoreType.DMA,
            pltpu.SemaphoreType.DMA,
        ],
    )


RB2 = 2048


def _mm_body(x_ref, cx_ref, w_ref, b_ref, degs_ref, degr_ref,
             h_ref, scale_r_ref):
    i = pl.program_id(0)
    scale_s = lax.rsqrt(jnp.maximum(degs_ref[...], 1.0))
    scale_r_ref[...] = lax.rsqrt(jnp.maximum(degr_ref[...], 1.0))

    @pl.when(i == 0)
    def _():
        h = jnp.dot(x_ref[...], w_ref[...], preferred_element_type=jnp.float32)
        h_ref[...] = (h + b_ref[...]) * scale_s

    @pl.when(i == 1)
    def _():
        h = jnp.dot(cx_ref[...], w_ref[...],
                    preferred_element_type=jnp.float32)
        h_ref[...] = (h + b_ref[...]) * scale_s


def _mm_kernel(x, c_x, W, b2, deg_s, deg_r):
    nj = NP // RB2
    return pl.pallas_call(
        _mm_body,
        grid=(2, nj),
        in_specs=[
            pl.BlockSpec((RB2, D), lambda i, j: (j, 0)),
            pl.BlockSpec((RB2, D), lambda i, j: (j, 0)),
            pl.BlockSpec((D, D), lambda i, j: (0, 0)),
            pl.BlockSpec((1, D), lambda i, j: (0, 0)),
            pl.BlockSpec((RB2, 1), lambda i, j: (j, 0)),
            pl.BlockSpec((RB2, 1), lambda i, j: (j, 0)),
        ],
        out_specs=[
            pl.BlockSpec((RB2, D), lambda i, j, nj=nj: (i * nj + j, 0)),
            pl.BlockSpec((RB2, 1), lambda i, j: (j, 0)),
        ],
        out_shape=[
            jax.ShapeDtypeStruct((2 * NP, D), jnp.float32),
            jax.ShapeDtypeStruct((NP, 1), jnp.float32),
        ],
    )(x, c_x, W, b2, deg_s, deg_r)


SELU_L = 1.0507009873554804934193349852946
SELU_A = 1.6732632423543772848170429916717


def _selu(a):
    return SELU_L * jnp.where(a > 0, a, SELU_A * (jnp.exp(a) - 1.0))


RB3 = 1024
NB3 = NP // RB3


def _head_body(a1_ref, ap_ref, scale_r_ref, w_blT_ref, centersT_ref,
               emb_ref, l1_ref, l2_ref, rep_ref, loss_ref,
               rowsum_s, v_s, csq_s, runmin_s, runarg_s, loss_s):
    p = pl.program_id(0)
    j = pl.program_id(1)

    scale_r = scale_r_ref[...]
    n1 = _selu(a1_ref[0] * scale_r)
    nrm = jnp.sqrt(jnp.sum(n1 * n1, axis=1, keepdims=True))
    emb = n1 / (nrm + 1e-12)
    emb_ref[...] = emb

    @pl.when(jnp.logical_and(p == 0, j == 0))
    def _():
        csq_s[...] = jnp.sum(centersT_ref[...] * centersT_ref[...], axis=0,
                             keepdims=True)
        rowsum_s[...] = jnp.zeros_like(rowsum_s)
        runmin_s[...] = jnp.full_like(runmin_s, jnp.inf)
        runarg_s[...] = jnp.zeros_like(runarg_s)
        loss_s[...] = jnp.zeros_like(loss_s)

    @pl.when(p == 0)
    def _():
        rowsum_s[...] += jnp.sum(n1, axis=0, keepdims=True)
        l1_ref[...] = jnp.zeros_like(l1_ref)
        l2_ref[...] = jnp.zeros_like(l2_ref)

        esq = jnp.sum(emb * emb, axis=1, keepdims=True)
        prod = jnp.dot(emb, centersT_ref[...],
                       preferred_element_type=jnp.float32)
        sq = esq + csq_s[...] - 2.0 * prod
        dists = jnp.sqrt(jnp.maximum(sq, 1e-12))

        row_local = lax.broadcasted_iota(jnp.int32, (RB3, K), 0)
        row_global = row_local + j * RB3
        valid = row_global < N
        dists_m = jnp.where(valid, dists, jnp.inf)

        dmin = jnp.min(dists_m, axis=1, keepdims=True)
        dmin = jnp.where(valid[:, :1], dmin, 0.0)
        loss_s[...] += jnp.sum(dmin).reshape(1, 1)

        bmin = jnp.min(dists_m, axis=0, keepdims=True)
        idxm = jnp.where(dists_m == bmin, row_global, 2**30)
        barg = jnp.min(idxm, axis=0, keepdims=True)
        better = bmin < runmin_s[...]
        runarg_s[...] = jnp.where(better, barg, runarg_s[...])
        runmin_s[...] = jnp.minimum(bmin, runmin_s[...])

        @pl.when(j == NB3 - 1)
        def _():
            rep_ref[...] = runarg_s[...]
            loss_ref[...] = loss_s[...]

    @pl.when(p == 1)
    def _():
        @pl.when(j == 0)
        def _():
            summary = 1.0 / (1.0 + jnp.exp(-rowsum_s[...] / N))
            v_s[...] = jnp.dot(summary, w_blT_ref[...],
                               preferred_element_type=jnp.float32)

        n2 = _selu(ap_ref[0] * scale_r)
        v = v_s[...]
        l1_ref[...] = jnp.sum(n1 * v, axis=1, keepdims=True)
        l2_ref[...] = jnp.sum(n2 * v, axis=1, keepdims=True)


def _head_kernel(agg, scale_r, w_blT, centersT):
    return pl.pallas_call(
        _head_body,
        grid=(2, NB3),
        in_specs=[
            pl.BlockSpec((1, RB3, D), lambda p, j: (0, j, 0)),
            pl.BlockSpec((1, RB3, D), lambda p, j: (p, j, 0)),
            pl.BlockSpec((RB3, 1), lambda p, j: (j, 0)),
            pl.BlockSpec((D, D), lambda p, j: (0, 0)),
            pl.BlockSpec((D, K), lambda p, j: (0, 0)),
        ],
        out_specs=[
            pl.BlockSpec((RB3, D), lambda p, j: (j, 0)),
            pl.BlockSpec((RB3, 1), lambda p, j: (j, 0)),
            pl.BlockSpec((RB3, 1), lambda p, j: (j, 0)),
            pl.BlockSpec((1, K), lambda p, j: (0, 0)),
            pl.BlockSpec((1, 1), lambda p, j: (0, 0)),
        ],
        out_shape=[
            jax.ShapeDtypeStruct((N, D), jnp.float32),
            jax.ShapeDtypeStruct((N, 1), jnp.float32),
            jax.ShapeDtypeStruct((N, 1), jnp.float32),
            jax.ShapeDtypeStruct((1, K), jnp.int32),
            jax.ShapeDtypeStruct((1, 1), jnp.float32),
        ],
        scratch_shapes=[
            pltpu.VMEM((1, D), jnp.float32),
            pltpu.VMEM((1, D), jnp.float32),
            pltpu.VMEM((1, K), jnp.float32),
            pltpu.VMEM((1, K), jnp.float32),
            pltpu.VMEM((1, K), jnp.int32),
            pltpu.VMEM((1, 1), jnp.float32),
        ],
    )(agg, agg, scale_r, w_blT, centersT)


def kernel(x, c_x, edge_index, W, b, w_bl, centers):
    ei = edge_index.astype(jnp.int32)
    e3 = ei.reshape(2, NS, NBLK, BB, ECH)

    e_deg = ei.reshape(2, NS, NCH3, CH)
    degp = _deg_kernel()(e_deg)

    h, scale_r = _mm_kernel(x, c_x, W, b.reshape(1, D),
                            degp[0].reshape(NP, 1), degp[1].reshape(NP, 1))
    agg = _edge_kernel()(h, e3)

    emb_p, l1, l2, rep, loss = _head_kernel(
        agg, scale_r, w_bl.T, centers.T)

    emb = emb_p
    logits = jnp.concatenate([l1[:, 0], l2[:, 0]])
    rep_ids = rep[0]
    cluster_loss = loss[0, 0]
    return (emb, centers, rep_ids, cluster_loss, logits)
